# Initial kernel scaffold; baseline (speedup 1.0000x reference)
import jax
import jax.numpy as jnp
from jax.experimental import pallas as pl


def kernel(x, edge_index, W_rel0, b_rel0, W_root0, W_rel1, b_rel1, W_root1, W_rel2, b_rel2, W_root2, W_rel3, b_rel3, W_root3, W_rel4, b_rel4, W_root4, W_rel5, b_rel5, W_root5):
    raise NotImplementedError("write your pallas kernel here")


# speedup vs baseline: 1.0001x; 1.0001x over previous
import jax
import jax.numpy as jnp
from jax.experimental import pallas as pl


def _sigmoid_kernel(z_ref, o_ref):
    o_ref[...] = jax.nn.sigmoid(z_ref[...])


def _graph_conv(x, src, dst, W_rel, b_rel, W_root, aggr):
    n = x.shape[0]
    msgs = x[src]
    if aggr == 'add':
        agg = jax.ops.segment_sum(msgs, dst, num_segments=n)
    elif aggr == 'mean':
        s = jax.ops.segment_sum(msgs, dst, num_segments=n)
        deg = jax.ops.segment_sum(jnp.ones((msgs.shape[0],), dtype=x.dtype), dst, num_segments=n)
        agg = s / jnp.clip(deg, 1.0)[:, None]
    else:
        agg = jax.ops.segment_max(msgs, dst, num_segments=n)
        agg = jnp.where(jnp.isfinite(agg), agg, 0.0)
    return agg @ W_rel + b_rel + x @ W_root


def kernel(x, edge_index, W_rel0, b_rel0, W_root0, W_rel1, b_rel1, W_root1, W_rel2, b_rel2, W_root2, W_rel3, b_rel3, W_root3, W_rel4, b_rel4, W_root4, W_rel5, b_rel5, W_root5):
    aggrs = ['add', 'mean', 'add', 'mean', 'max', 'max']
    Ws = [(W_rel0, b_rel0, W_root0), (W_rel1, b_rel1, W_root1), (W_rel2, b_rel2, W_root2),
          (W_rel3, b_rel3, W_root3), (W_rel4, b_rel4, W_root4), (W_rel5, b_rel5, W_root5)]
    src = edge_index[0]
    dst = edge_index[1]
    h = x.reshape(-1, 1)
    for i, aggr in enumerate(aggrs):
        Wr, br, Wo = Ws[i]
        h = _graph_conv(h, src, dst, Wr, br, Wo, aggr)
        if i < len(aggrs) - 1:
            h = jax.nn.relu(h)
    z = h.reshape(-1)
    out = pl.pallas_call(
        _sigmoid_kernel,
        out_shape=jax.ShapeDtypeStruct(z.shape, z.dtype),
    )(z)
    return out


# speedup vs baseline: 4.0073x; 4.0070x over previous
import functools

import jax
import jax.numpy as jnp
from jax import lax
from jax.experimental import pallas as pl
from jax.experimental.pallas import tpu as pltpu
from jax.experimental.pallas import tpu_sc as plsc

N = 100000
E = 1600000
ROWS = E // 128
UNITS = ROWS // 16
REM_ROW = UNITS * 16
NT = 16
NC = 2
NW = NT * NC
NB = 32
BUCKET = 3128
SLICE = 2 * BUCKET
LAST = N - 15 * SLICE
EP = E + 128

_mesh = plsc.VectorSubcoreMesh(core_axis_name="c", subcore_axis_name="s")


def _bucket_of(d):
    return (d.astype(jnp.float32) / float(BUCKET)).astype(jnp.int32)


def _tile_rows(t):
    return t * SLICE, (LAST if t == NT - 1 else SLICE)


def _copy_tile_slice(t, src, dst):
    for ts in range(NT):
        start, size = _tile_rows(ts)

        @pl.when(t == ts)
        def _(start=start, size=size):
            pltpu.sync_copy(src.at[pl.ds(start, size)], dst.at[pl.ds(start, size)])


def _l0_body(tab0, src2d, dst2d, zeros2, part, hist, acc, sbufs, dbufs, msgs,
             histv, gsem):
    c = lax.axis_index("c")
    t = lax.axis_index("s")
    w = t * NC + c
    iota = lax.iota(jnp.int32, 16)
    ones = jnp.ones((16,), jnp.int32)

    _copy_tile_slice(t, zeros2, acc)

    def zh(i, _):
        histv[0, pl.ds(i * 16, 16)] = jnp.zeros((16,), jnp.int32)
        return _

    lax.fori_loop(0, NB, zh, 0)
    plsc.subcore_barrier()

    def hist_batch(nrows):
        def hrow(j, _):
            def hvec(v, _):
                d = dbufs[j, pl.ds(v * 16, 16)]
                b = _bucket_of(d)
                plsc.addupdate_scatter(histv.at[0], [b * 16 + iota], ones)
                return _

            return lax.fori_loop(0, 8, hvec, _)

        lax.fori_loop(0, nrows, hrow, 0)

    def do_batch(r, nrows):
        pltpu.sync_copy(src2d.at[pl.ds(r, nrows)], sbufs.at[pl.ds(0, nrows)])
        pltpu.sync_copy(dst2d.at[pl.ds(r, nrows)], dbufs.at[pl.ds(0, nrows)])
        for h in range(0, nrows, 8):
            hb = min(8, nrows - h)
            cps = [pltpu.async_copy(tab0.at[sbufs.at[h + j]], msgs.at[j], gsem)
                   for j in range(hb)]
            for cp in cps:
                cp.wait()
            for j in range(hb):
                pltpu.sync_copy(msgs.at[j], acc.at[dbufs.at[h + j]], add=True)
        hist_batch(nrows)

    lo_u = (w * UNITS) // NW
    hi_u = ((w + 1) * UNITS) // NW

    def batch(k, _):
        do_batch(k * 16, 16)
        return _

    lax.fori_loop(lo_u, hi_u, batch, 0)

    @pl.when(w == NW - 1)
    def _rem():
        do_batch(REM_ROW, 4)

    plsc.subcore_barrier()
    for ts in range(NT):
        start, size = _tile_rows(ts)

        @pl.when(t == ts)
        def _(start=start, size=size):
            pltpu.sync_copy(acc.at[pl.ds(start, size)],
                            part.at[c, pl.ds(start, size)])
    pltpu.sync_copy(histv, hist.at[w])


def _b2_body(src2d, dst2d, hist, bsrc, bdst, histv, curs, sbufs, dbufs, pbufs,
             s1, s2):
    c = lax.axis_index("c")
    t = lax.axis_index("s")
    w = t * NC + c
    iota = lax.iota(jnp.int32, 16)
    ones = jnp.ones((16,), jnp.int32)
    zeros16i = jnp.zeros((16,), jnp.int32)

    pltpu.sync_copy(hist, histv)

    base_acc = jnp.int32(0)
    for b in range(NB):
        def inner(w2, carry, b=b):
            pre, tot = carry
            row = histv[w2, 0, pl.ds(b * 16, 16)]
            pre = pre + jnp.where(w2 < w, row, zeros16i)
            tot = tot + row
            return (pre, tot)

        pre, tot = lax.fori_loop(0, NW, inner, (zeros16i, zeros16i))
        own = histv[w, 0, pl.ds(b * 16, 16)]
        ownprefix = plsc.cumsum(own) - own
        curs[pl.ds(b * 16, 16)] = base_acc + jnp.sum(pre) + ownprefix
        base_acc = base_acc + jnp.sum(tot)

    def do_batch(r, nrows):
        pltpu.sync_copy(src2d.at[pl.ds(r, nrows)], sbufs.at[pl.ds(0, nrows)])
        pltpu.sync_copy(dst2d.at[pl.ds(r, nrows)], dbufs.at[pl.ds(0, nrows)])
        for j in range(nrows):
            for v in range(8):
                d = dbufs[j, pl.ds(v * 16, 16)]
                idx = _bucket_of(d) * 16 + iota
                pbufs[j, pl.ds(v * 16, 16)] = plsc.load_gather(curs, [idx])
                plsc.addupdate_scatter(curs, [idx], ones)
        cps = []
        for j in range(nrows):
            cps.append(pltpu.async_copy(sbufs.at[j], bsrc.at[pbufs.at[j]], s1))
            cps.append(pltpu.async_copy(dbufs.at[j], bdst.at[pbufs.at[j]], s2))
        for cp in cps:
            cp.wait()

    lo_u = (w * UNITS) // NW
    hi_u = ((w + 1) * UNITS) // NW

    def batch(k, _):
        do_batch(k * 16, 16)
        return _

    lax.fori_loop(lo_u, hi_u, batch, 0)

    @pl.when(w == NW - 1)
    def _rem():
        do_batch(REM_ROW, 4)

    @pl.when(w == 0)
    def _pad():
        for v in range(8):
            pbufs[0, pl.ds(v * 16, 16)] = jnp.zeros((16,), jnp.int32)
        pltpu.sync_copy(pbufs.at[0], bsrc.at[pl.ds(E, 128)])
        pltpu.sync_copy(pbufs.at[0], bdst.at[pl.ds(E, 128)])


def _sum_body(hA, hB, src2d, dst2d, zeros16, sA, sB, acc, sbufs, dbufs, msgs,
              gsem):
    c = lax.axis_index("c")
    t = lax.axis_index("s")

    _copy_tile_slice(t, zeros16, acc)
    plsc.subcore_barrier()

    lo_u = (t * UNITS) // NT
    hi_u = ((t + 1) * UNITS) // NT

    def run(tab):
        def do_batch(r, nrows):
            pltpu.sync_copy(src2d.at[pl.ds(r, nrows)], sbufs.at[pl.ds(0, nrows)])
            pltpu.sync_copy(dst2d.at[pl.ds(r, nrows)], dbufs.at[pl.ds(0, nrows)])
            for h in range(0, nrows, 8):
                hb = min(8, nrows - h)
                cps = [pltpu.async_copy(tab.at[sbufs.at[h + j]], msgs.at[j],
                                        gsem)
                       for j in range(hb)]
                for cp in cps:
                    cp.wait()
                for j in range(hb):
                    pltpu.sync_copy(msgs.at[j], acc.at[dbufs.at[h + j]],
                                    add=True)

        def batch(k, _):
            do_batch(k * 16, 16)
            return _

        lax.fori_loop(lo_u, hi_u, batch, 0)

        @pl.when(t == NT - 1)
        def _rem():
            do_batch(REM_ROW, 4)

    @pl.when(c == 0)
    def _a():
        run(hA)

    @pl.when(c == 1)
    def _b():
        run(hB)

    plsc.subcore_barrier()
    for ts in range(NT):
        start, size = _tile_rows(ts)

        @pl.when(t == ts)
        def _(start=start, size=size):
            @pl.when(c == 0)
            def _wa():
                pltpu.sync_copy(acc.at[pl.ds(start, size)],
                                sA.at[pl.ds(start, size)])

            @pl.when(c == 1)
            def _wb():
                pltpu.sync_copy(acc.at[pl.ds(start, size)],
                                sB.at[pl.ds(start, size)])


def _max_body(hA, hB, bsrc, bdst, hist, mA, mB, acc, histv, sbuf, dbuf, msgs,
              gsem):
    c = lax.axis_index("c")
    t = lax.axis_index("s")

    pltpu.sync_copy(hist, histv)

    def outer(w2, carry):
        def inner(q, carry):
            base, cnt = carry
            v = jnp.sum(histv[w2, 0, pl.ds(q * 16, 16)])
            base = base + jnp.where(q < 2 * t, v, 0)
            cnt = cnt + jnp.where((q == 2 * t) | (q == 2 * t + 1), v, 0)
            return (base, cnt)

        return lax.fori_loop(0, NB, inner, carry)

    base, cnt = lax.fori_loop(0, NW, outer, (jnp.int32(0), jnp.int32(0)))

    def za(i, _):
        acc[i, :] = jnp.zeros((16,), jnp.float32)
        return _

    lax.fori_loop(0, SLICE, za, 0)

    blk0 = base // 128
    off0 = base - blk0 * 128
    nblk = (off0 + cnt + 127) // 128

    def run(tab):
        def blk(k, _):
            g = (blk0 + k) * 128
            pltpu.sync_copy(bsrc.at[pl.ds(g, 128)], sbuf)
            pltpu.sync_copy(bdst.at[pl.ds(g, 128)], dbuf.at[pl.ds(0, 128)])
            pltpu.async_copy(tab.at[sbuf], msgs, gsem).wait()
            e_lo = jnp.maximum(base - g, 0)
            e_hi = jnp.minimum(base + cnt - g, 128)

            def edge(e, _):
                dl = dbuf[pl.ds(e, 16)][0] - t * SLICE
                acc[dl, :] = jnp.maximum(acc[dl, :], msgs[e, :])
                return _

            lax.fori_loop(e_lo, e_hi, edge, 0)
            return _

        lax.fori_loop(0, nblk, blk, 0)

    @pl.when(c == 0)
    def _a():
        run(hA)

    @pl.when(c == 1)
    def _b():
        run(hB)

    for ts in range(NT):
        start, size = _tile_rows(ts)

        @pl.when(t == ts)
        def _(start=start, size=size):
            @pl.when(c == 0)
            def _wa():
                pltpu.sync_copy(acc.at[pl.ds(0, size)],
                                mA.at[pl.ds(start, size)])

            @pl.when(c == 1)
            def _wb():
                pltpu.sync_copy(acc.at[pl.ds(0, size)],
                                mB.at[pl.ds(start, size)])


_SC_PARAMS = pltpu.CompilerParams(needs_layout_passes=False, use_tc_tiling_on_sc=False)

_l0_kernel = functools.partial(
    pl.kernel,
    mesh=_mesh,
    compiler_params=_SC_PARAMS,
    out_type=[
        jax.ShapeDtypeStruct((NC, N, 8), jnp.float32),
        jax.ShapeDtypeStruct((NW, 1, NB * 16), jnp.int32),
    ],
    scratch_types=[
        pltpu.VMEM_SHARED((N, 8), jnp.float32),
        pltpu.VMEM((16, 128), jnp.int32),
        pltpu.VMEM((16, 128), jnp.int32),
        pltpu.VMEM((16, 128, 8), jnp.float32),
        pltpu.VMEM((1, NB * 16), jnp.int32),
        pltpu.SemaphoreType.DMA,
    ],
)(_l0_body)

_b2_kernel = functools.partial(
    pl.kernel,
    mesh=_mesh,
    compiler_params=_SC_PARAMS,
    out_type=[
        jax.ShapeDtypeStruct((EP,), jnp.int32),
        jax.ShapeDtypeStruct((EP,), jnp.int32),
    ],
    scratch_types=[
        pltpu.VMEM((NW, 1, NB * 16), jnp.int32),
        pltpu.VMEM((NB * 16,), jnp.int32),
        pltpu.VMEM((16, 128), jnp.int32),
        pltpu.VMEM((16, 128), jnp.int32),
        pltpu.VMEM((16, 128), jnp.int32),
        pltpu.SemaphoreType.DMA,
        pltpu.SemaphoreType.DMA,
    ],
)(_b2_body)

_sum_kernel = functools.partial(
    pl.kernel,
    mesh=_mesh,
    compiler_params=_SC_PARAMS,
    out_type=[
        jax.ShapeDtypeStruct((N, 16), jnp.float32),
        jax.ShapeDtypeStruct((N, 16), jnp.float32),
    ],
    scratch_types=[
        pltpu.VMEM_SHARED((N, 16), jnp.float32),
        pltpu.VMEM((16, 128), jnp.int32),
        pltpu.VMEM((16, 128), jnp.int32),
        pltpu.VMEM((8, 128, 16), jnp.float32),
        pltpu.SemaphoreType.DMA,
    ],
)(_sum_body)

_max_kernel = functools.partial(
    pl.kernel,
    mesh=_mesh,
    compiler_params=_SC_PARAMS,
    out_type=[
        jax.ShapeDtypeStruct((N, 16), jnp.float32),
        jax.ShapeDtypeStruct((N, 16), jnp.float32),
    ],
    scratch_types=[
        pltpu.VMEM((SLICE, 16), jnp.float32),
        pltpu.VMEM((NW, 1, NB * 16), jnp.int32),
        pltpu.VMEM((128,), jnp.int32),
        pltpu.VMEM((144,), jnp.int32),
        pltpu.VMEM((128, 16), jnp.float32),
        pltpu.SemaphoreType.DMA,
    ],
)(_max_body)


_RB = 5000
_GRID = N // _RB


def _dense0_body(x_ref, p0_ref, p1_ref, wr_ref, b_ref, wo_ref,
                 ha_ref, hb_ref, rec_ref):
    s = p0_ref[:, 0:1] + p1_ref[:, 0:1]
    deg = p0_ref[:, 1:2] + p1_ref[:, 1:2]
    h = (s * wr_ref[0:1, :] + b_ref[...][None, :]
         + x_ref[...] * wo_ref[0:1, :])
    h = jnp.maximum(h, 0.0)
    ha_ref[...] = h[:, :16]
    hb_ref[...] = h[:, 16:]
    rec_ref[...] = 1.0 / jnp.maximum(deg, 1.0)


def _dense0(x1, p0, p1, wr, b, wo):
    return pl.pallas_call(
        _dense0_body,
        grid=(_GRID,),
        in_specs=[
            pl.BlockSpec((_RB, 1), lambda i: (i, 0)),
            pl.BlockSpec((_RB, 8), lambda i: (i, 0)),
            pl.BlockSpec((_RB, 8), lambda i: (i, 0)),
            pl.BlockSpec((1, 32), lambda i: (0, 0)),
            pl.BlockSpec((32,), lambda i: (0,)),
            pl.BlockSpec((1, 32), lambda i: (0, 0)),
        ],
        out_specs=[
            pl.BlockSpec((_RB, 16), lambda i: (i, 0)),
            pl.BlockSpec((_RB, 16), lambda i: (i, 0)),
            pl.BlockSpec((_RB, 1), lambda i: (i, 0)),
        ],
        out_shape=[
            jax.ShapeDtypeStruct((N, 16), jnp.float32),
            jax.ShapeDtypeStruct((N, 16), jnp.float32),
            jax.ShapeDtypeStruct((N, 1), jnp.float32),
        ],
    )(x1, p0, p1, wr, b, wo)


def _dense_mid_body(use_mean, relu, sa_ref, sb_ref, ha_ref, hb_ref, wr_ref,
                    b_ref, wo_ref, rec_ref, za_ref, zb_ref):
    s = jnp.concatenate([sa_ref[...], sb_ref[...]], axis=1)
    if use_mean:
        s = s * rec_ref[...]
    h = jnp.concatenate([ha_ref[...], hb_ref[...]], axis=1)
    z = (jnp.dot(s, wr_ref[...], preferred_element_type=jnp.float32)
         + b_ref[...][None, :]
         + jnp.dot(h, wo_ref[...], preferred_element_type=jnp.float32))
    if relu:
        z = jnp.maximum(z, 0.0)
    za_ref[...] = z[:, :16]
    zb_ref[...] = z[:, 16:]


def _dense_mid(sa, sb, ha, hb, wr, b, wo, rec, use_mean, relu=True):
    return pl.pallas_call(
        functools.partial(_dense_mid_body, use_mean, relu),
        grid=(_GRID,),
        in_specs=[
            pl.BlockSpec((_RB, 16), lambda i: (i, 0)),
            pl.BlockSpec((_RB, 16), lambda i: (i, 0)),
            pl.BlockSpec((_RB, 16), lambda i: (i, 0)),
            pl.BlockSpec((_RB, 16), lambda i: (i, 0)),
            pl.BlockSpec((32, 32), lambda i: (0, 0)),
            pl.BlockSpec((32,), lambda i: (0,)),
            pl.BlockSpec((32, 32), lambda i: (0, 0)),
            pl.BlockSpec((_RB, 1), lambda i: (i, 0)),
        ],
        out_specs=[
            pl.BlockSpec((_RB, 16), lambda i: (i, 0)),
            pl.BlockSpec((_RB, 16), lambda i: (i, 0)),
        ],
        out_shape=[
            jax.ShapeDtypeStruct((N, 16), jnp.float32),
            jax.ShapeDtypeStruct((N, 16), jnp.float32),
        ],
    )(sa, sb, ha, hb, wr, b, wo, rec)


def _dense_final_body(sa_ref, sb_ref, ha_ref, hb_ref, wr_ref, b_ref, wo_ref,
                      o_ref):
    s = jnp.concatenate([sa_ref[...], sb_ref[...]], axis=1)
    h = jnp.concatenate([ha_ref[...], hb_ref[...]], axis=1)
    z = (jnp.dot(s, wr_ref[...], preferred_element_type=jnp.float32)
         + b_ref[...][None, :]
         + jnp.dot(h, wo_ref[...], preferred_element_type=jnp.float32))
    o_ref[...] = jax.nn.sigmoid(z)


def _dense_final(sa, sb, ha, hb, wr, b, wo):
    return pl.pallas_call(
        _dense_final_body,
        grid=(_GRID,),
        in_specs=[
            pl.BlockSpec((_RB, 16), lambda i: (i, 0)),
            pl.BlockSpec((_RB, 16), lambda i: (i, 0)),
            pl.BlockSpec((_RB, 16), lambda i: (i, 0)),
            pl.BlockSpec((_RB, 16), lambda i: (i, 0)),
            pl.BlockSpec((32, 1), lambda i: (0, 0)),
            pl.BlockSpec((1,), lambda i: (0,)),
            pl.BlockSpec((32, 1), lambda i: (0, 0)),
        ],
        out_specs=pl.BlockSpec((_RB, 1), lambda i: (i, 0)),
        out_shape=jax.ShapeDtypeStruct((N, 1), jnp.float32),
    )(sa, sb, ha, hb, wr, b, wo)


def kernel(x, edge_index, W_rel0, b_rel0, W_root0, W_rel1, b_rel1, W_root1,
           W_rel2, b_rel2, W_root2, W_rel3, b_rel3, W_root3, W_rel4, b_rel4,
           W_root4, W_rel5, b_rel5, W_root5):
    src2d = edge_index[0].reshape(ROWS, 128)
    dst2d = edge_index[1].reshape(ROWS, 128)
    tab0 = jnp.concatenate(
        [x[:, None], jnp.ones((N, 1), jnp.float32),
         jnp.zeros((N, 6), jnp.float32)], axis=1)
    zeros16 = jnp.zeros((N, 16), jnp.float32)
    zeros2 = jnp.zeros((N, 8), jnp.float32)

    part, hist = _l0_kernel(tab0, src2d, dst2d, zeros2)
    bsrc, bdst = _b2_kernel(src2d, dst2d, hist)

    h1a, h1b, rec = _dense0(x.reshape(N, 1), part[0], part[1],
                            W_rel0, b_rel0, W_root0)

    s1a, s1b = _sum_kernel(h1a, h1b, src2d, dst2d, zeros16)
    h2a, h2b = _dense_mid(s1a, s1b, h1a, h1b, W_rel1, b_rel1, W_root1, rec,
                          use_mean=True)

    s2a, s2b = _sum_kernel(h2a, h2b, src2d, dst2d, zeros16)
    h3a, h3b = _dense_mid(s2a, s2b, h2a, h2b, W_rel2, b_rel2, W_root2, rec,
                          use_mean=False)

    s3a, s3b = _sum_kernel(h3a, h3b, src2d, dst2d, zeros16)
    h4a, h4b = _dense_mid(s3a, s3b, h3a, h3b, W_rel3, b_rel3, W_root3, rec,
                          use_mean=True)

    m4a, m4b = _max_kernel(h4a, h4b, bsrc, bdst, hist)
    h5a, h5b = _dense_mid(m4a, m4b, h4a, h4b, W_rel4, b_rel4, W_root4, rec,
                          use_mean=False)

    m5a, m5b = _max_kernel(h5a, h5b, bsrc, bdst, hist)
    out = _dense_final(m5a, m5b, h5a, h5b, W_rel5, b_rel5, W_root5)
    return out.reshape(N)


# speedup vs baseline: 7.3607x; 1.8368x over previous
import functools

import jax
import jax.numpy as jnp
from jax import lax
from jax.experimental import pallas as pl
from jax.experimental.pallas import tpu as pltpu
from jax.experimental.pallas import tpu_sc as plsc

N = 100000
E = 1600000
ROWS = E // 128
UNITS = ROWS // 16
REM_ROW = UNITS * 16
NT = 16
NC = 2
NW = NT * NC
NB = 32
BUCKET = 3128
LASTB = N - 31 * BUCKET
SLICE = 2 * BUCKET
LAST = N - 15 * SLICE
EP = E + 512
PBITS = 12

_mesh = plsc.VectorSubcoreMesh(core_axis_name="c", subcore_axis_name="s")
_SC_PARAMS = pltpu.CompilerParams(needs_layout_passes=False,
                                  use_tc_tiling_on_sc=False)


def _bucket_of(d):
    return (d.astype(jnp.float32) / float(BUCKET)).astype(jnp.int32)


def _tile_rows(t):
    return t * SLICE, (LAST if t == NT - 1 else SLICE)


def _copy_tile_slice(t, src, dst):
    for ts in range(NT):
        start, size = _tile_rows(ts)

        @pl.when(t == ts)
        def _(start=start, size=size):
            pltpu.sync_copy(src.at[pl.ds(start, size)], dst.at[pl.ds(start, size)])


def _unit_range(i, n):
    return (i * UNITS) // n, ((i + 1) * UNITS) // n


def _l0_body(tab0, src2d, dst2d, zeros8, part, hist, acc, sbufs, dbufs, msgsA,
             msgsB, histv, gsem, ssem):
    c = lax.axis_index("c")
    t = lax.axis_index("s")
    w = t * NC + c
    iota = lax.iota(jnp.int32, 16)
    ones = jnp.ones((16,), jnp.int32)

    _copy_tile_slice(t, zeros8, acc)

    def zh(i, _):
        histv[0, pl.ds(i * 16, 16)] = jnp.zeros((16,), jnp.int32)
        return _

    lax.fori_loop(0, NB, zh, 0)
    plsc.subcore_barrier()

    def hist_rows(nrows):
        def hrow(j, _):
            def hvec(v, _):
                d = dbufs[j, pl.ds(v * 16, 16)]
                b = _bucket_of(d)
                plsc.addupdate_scatter(histv.at[0], [b * 16 + iota], ones)
                return _

            return lax.fori_loop(0, 8, hvec, _)

        lax.fori_loop(0, nrows, hrow, 0)

    def do_batch(r, nrows):
        pltpu.sync_copy(src2d.at[pl.ds(r, nrows)], sbufs.at[pl.ds(0, nrows)])
        pltpu.sync_copy(dst2d.at[pl.ds(r, nrows)], dbufs.at[pl.ds(0, nrows)])
        na = min(8, nrows)
        ga = [pltpu.async_copy(tab0.at[sbufs.at[j]], msgsA.at[j], gsem)
              for j in range(na)]
        for cp in ga:
            cp.wait()
        sa = [pltpu.async_copy(msgsA.at[j], acc.at[dbufs.at[j]], ssem,
                               add=True)
              for j in range(na)]
        sb = []
        if nrows > 8:
            gb = [pltpu.async_copy(tab0.at[sbufs.at[8 + j]], msgsB.at[j], gsem)
                  for j in range(nrows - 8)]
            for cp in gb:
                cp.wait()
            sb = [pltpu.async_copy(msgsB.at[j], acc.at[dbufs.at[8 + j]], ssem,
                                   add=True)
                  for j in range(nrows - 8)]
        hist_rows(nrows)
        for cp in sa + sb:
            cp.wait()

    lo_u, hi_u = _unit_range(w, NW)

    def batch(k, _):
        do_batch(k * 16, 16)
        return _

    lax.fori_loop(lo_u, hi_u, batch, 0)

    @pl.when(w == NW - 1)
    def _rem():
        do_batch(REM_ROW, 4)

    plsc.subcore_barrier()
    for ts in range(NT):
        start, size = _tile_rows(ts)

        @pl.when(t == ts)
        def _(start=start, size=size):
            pltpu.sync_copy(acc.at[pl.ds(start, size)],
                            part.at[c, pl.ds(start, size)])
    pltpu.sync_copy(histv, hist.at[w])


_l0_kernel = functools.partial(
    pl.kernel,
    mesh=_mesh,
    compiler_params=_SC_PARAMS,
    out_type=[
        jax.ShapeDtypeStruct((NC, N, 8), jnp.float32),
        jax.ShapeDtypeStruct((NW, 1, NB * 16), jnp.int32),
    ],
    scratch_types=[
        pltpu.VMEM_SHARED((N, 8), jnp.float32),
        pltpu.VMEM((16, 128), jnp.int32),
        pltpu.VMEM((16, 128), jnp.int32),
        pltpu.VMEM((8, 128, 8), jnp.float32),
        pltpu.VMEM((8, 128, 8), jnp.float32),
        pltpu.VMEM((1, NB * 16), jnp.int32),
        pltpu.SemaphoreType.DMA,
        pltpu.SemaphoreType.DMA,
    ],
)(_l0_body)


def _b2_body(src2d, dst2d, hist, bpack, btot, histv, curs, sbufs, dbufs, packv,
             posv, totv, s1):
    c = lax.axis_index("c")
    t = lax.axis_index("s")
    w = t * NC + c
    iota = lax.iota(jnp.int32, 16)
    ones = jnp.ones((16,), jnp.int32)
    zeros16i = jnp.zeros((16,), jnp.int32)

    pltpu.sync_copy(hist, histv)

    base_acc = jnp.int32(0)
    for b in range(NB):
        def inner(w2, carry, b=b):
            pre, tot = carry
            row = histv[w2, 0, pl.ds(b * 16, 16)]
            pre = pre + jnp.where(w2 < w, row, zeros16i)
            tot = tot + row
            return (pre, tot)

        pre, tot = lax.fori_loop(0, NW, inner, (zeros16i, zeros16i))
        own = histv[w, 0, pl.ds(b * 16, 16)]
        ownprefix = plsc.cumsum(own) - own
        curs[pl.ds(b * 16, 16)] = base_acc + jnp.sum(pre) + ownprefix
        base_acc = base_acc + jnp.sum(tot)

        @pl.when(w == 0)
        def _(b=b, tot=tot):
            totv[0, pl.ds(b * 16, 16)] = tot

    @pl.when(w == 0)
    def _tot():
        pltpu.sync_copy(totv, btot)

    def do_batch(r, nrows):
        pltpu.sync_copy(src2d.at[pl.ds(r, nrows)], sbufs.at[pl.ds(0, nrows)])
        pltpu.sync_copy(dst2d.at[pl.ds(r, nrows)], dbufs.at[pl.ds(0, nrows)])
        cps = []
        for j in range(nrows):
            for v in range(8):
                d = dbufs[j, pl.ds(v * 16, 16)]
                s = sbufs[j, pl.ds(v * 16, 16)]
                b = _bucket_of(d)
                idx = b * 16 + iota
                pos = plsc.load_gather(curs, [idx])
                plsc.addupdate_scatter(curs, [idx], ones)
                posv[j, pl.ds(v * 16, 16)] = pos
                packv[j, pl.ds(v * 16, 16)] = (s << PBITS) | (d - b * BUCKET)
            cps.append(pltpu.async_copy(packv.at[j], bpack.at[posv.at[j]], s1))
        for cp in cps:
            cp.wait()

    lo_u, hi_u = _unit_range(w, NW)

    def batch(k, _):
        do_batch(k * 16, 16)
        return _

    lax.fori_loop(lo_u, hi_u, batch, 0)

    @pl.when(w == NW - 1)
    def _rem():
        do_batch(REM_ROW, 4)

    @pl.when(w == 0)
    def _pad():
        for j in range(4):
            for v in range(8):
                packv[j, pl.ds(v * 16, 16)] = jnp.zeros((16,), jnp.int32)
        for j in range(4):
            pltpu.sync_copy(packv.at[j], bpack.at[pl.ds(E + j * 128, 128)])


_b2_kernel = functools.partial(
    pl.kernel,
    mesh=_mesh,
    compiler_params=_SC_PARAMS,
    out_type=[
        jax.ShapeDtypeStruct((EP,), jnp.int32),
        jax.ShapeDtypeStruct((1, NB * 16), jnp.int32),
    ],
    scratch_types=[
        pltpu.VMEM((NW, 1, NB * 16), jnp.int32),
        pltpu.VMEM((NB * 16,), jnp.int32),
        pltpu.VMEM((16, 128), jnp.int32),
        pltpu.VMEM((16, 128), jnp.int32),
        pltpu.VMEM((16, 128), jnp.int32),
        pltpu.VMEM((16, 128), jnp.int32),
        pltpu.VMEM((1, NB * 16), jnp.int32),
        pltpu.SemaphoreType.DMA,
    ],
)(_b2_body)


def _sum_body(hA, hB, src2d, dst2d, zeros16, sA, sB, acc, sbufs, dbufs, msgsA,
              msgsB, gsem, ssem):
    c = lax.axis_index("c")
    t = lax.axis_index("s")

    _copy_tile_slice(t, zeros16, acc)
    plsc.subcore_barrier()

    def run(tab):
        def do_batch(r, nrows):
            pltpu.sync_copy(src2d.at[pl.ds(r, nrows)], sbufs.at[pl.ds(0, nrows)])
            pltpu.sync_copy(dst2d.at[pl.ds(r, nrows)], dbufs.at[pl.ds(0, nrows)])
            bufs = [msgsA, msgsB]
            pend = [[], []]
            for wv in range(nrows // 4):
                bi = wv % 2
                for cp in pend[bi]:
                    cp.wait()
                g = [pltpu.async_copy(tab.at[sbufs.at[wv * 4 + j]],
                                      bufs[bi].at[j], gsem)
                     for j in range(4)]
                for cp in g:
                    cp.wait()
                pend[bi] = [pltpu.async_copy(bufs[bi].at[j],
                                             acc.at[dbufs.at[wv * 4 + j]],
                                             ssem, add=True)
                            for j in range(4)]
            for cp in pend[0] + pend[1]:
                cp.wait()

        lo_u, hi_u = _unit_range(t, NT)

        def batch(k, _):
            do_batch(k * 16, 16)
            return _

        lax.fori_loop(lo_u, hi_u, batch, 0)

        @pl.when(t == NT - 1)
        def _rem():
            do_batch(REM_ROW, 4)

    @pl.when(c == 0)
    def _a():
        run(hA)

    @pl.when(c == 1)
    def _b():
        run(hB)

    plsc.subcore_barrier()
    for ts in range(NT):
        start, size = _tile_rows(ts)

        @pl.when(t == ts)
        def _(start=start, size=size):
            @pl.when(c == 0)
            def _wa():
                pltpu.sync_copy(acc.at[pl.ds(start, size)],
                                sA.at[pl.ds(start, size)])

            @pl.when(c == 1)
            def _wb():
                pltpu.sync_copy(acc.at[pl.ds(start, size)],
                                sB.at[pl.ds(start, size)])


_sum_kernel = functools.partial(
    pl.kernel,
    mesh=_mesh,
    compiler_params=_SC_PARAMS,
    out_type=[
        jax.ShapeDtypeStruct((N, 16), jnp.float32),
        jax.ShapeDtypeStruct((N, 16), jnp.float32),
    ],
    scratch_types=[
        pltpu.VMEM_SHARED((N, 16), jnp.float32),
        pltpu.VMEM((16, 128), jnp.int32),
        pltpu.VMEM((16, 128), jnp.int32),
        pltpu.VMEM((4, 128, 16), jnp.float32),
        pltpu.VMEM((4, 128, 16), jnp.float32),
        pltpu.SemaphoreType.DMA,
        pltpu.SemaphoreType.DMA,
    ],
)(_sum_body)


MBLK = 512


def _max_body(h32, bpack, btot, m32, acc, totv, sbufs, dlv, pbuf, msgs, gsem):
    c = lax.axis_index("c")
    t = lax.axis_index("s")
    w = t * NC + c
    mask = jnp.int32((1 << PBITS) - 1)

    pltpu.sync_copy(btot, totv)

    def fold(q, carry):
        base, cnt = carry
        v = jnp.sum(totv[0, pl.ds(q * 16, 16)])
        base = base + jnp.where(q < w, v, 0)
        cnt = cnt + jnp.where(q == w, v, 0)
        return (base, cnt)

    base, cnt = lax.fori_loop(0, NB, fold, (jnp.int32(0), jnp.int32(0)))

    def za(i, _):
        acc[i, pl.ds(0, 16)] = jnp.zeros((16,), jnp.float32)
        acc[i, pl.ds(16, 16)] = jnp.zeros((16,), jnp.float32)
        return _

    lax.fori_loop(0, BUCKET, za, 0)

    blk0 = base // MBLK
    off0 = base - blk0 * MBLK
    nblk = (off0 + cnt + MBLK - 1) // MBLK

    def blk(k, _):
        g = (blk0 + k) * MBLK
        pltpu.sync_copy(bpack.at[pl.ds(g, MBLK)], pbuf)

        def unpack(j2, _):
            def up(v, _):
                p = pbuf[pl.ds(j2 * 128 + v * 16, 16)]
                sbufs[j2, pl.ds(v * 16, 16)] = lax.shift_right_logical(p, PBITS)
                dlv[pl.ds(j2 * 128 + v * 16, 16)] = p & mask
                return _

            return lax.fori_loop(0, 8, up, _)

        lax.fori_loop(0, MBLK // 128, unpack, 0)
        cps = [pltpu.async_copy(h32.at[sbufs.at[j]],
                                msgs.at[pl.ds(j * 128, 128)], gsem)
               for j in range(MBLK // 128)]
        for cp in cps:
            cp.wait()
        e_lo = jnp.maximum(base - g, 0)
        e_hi = jnp.minimum(base + cnt - g, MBLK)

        def one_edge(e):
            dl = dlv[pl.ds(e, 16)][0]
            acc[dl, pl.ds(0, 16)] = jnp.maximum(acc[dl, pl.ds(0, 16)],
                                                msgs[e, pl.ds(0, 16)])
            acc[dl, pl.ds(16, 16)] = jnp.maximum(acc[dl, pl.ds(16, 16)],
                                                 msgs[e, pl.ds(16, 16)])

        def body8(e):
            for u in range(8):
                one_edge(e + u)
            return e + 8

        e_mid = lax.while_loop(lambda e: e + 8 <= e_hi, body8, e_lo)

        def edge(e, _):
            one_edge(e)
            return _

        lax.fori_loop(e_mid, e_hi, edge, 0)
        return _

    lax.fori_loop(0, nblk, blk, 0)

    @pl.when(w < NW - 1)
    def _wr():
        pltpu.sync_copy(acc.at[pl.ds(0, BUCKET)],
                        m32.at[pl.ds(w * BUCKET, BUCKET)])

    @pl.when(w == NW - 1)
    def _wl():
        pltpu.sync_copy(acc.at[pl.ds(0, LASTB)],
                        m32.at[pl.ds((NW - 1) * BUCKET, LASTB)])


_max_kernel = functools.partial(
    pl.kernel,
    mesh=_mesh,
    compiler_params=_SC_PARAMS,
    out_type=jax.ShapeDtypeStruct((N, 32), jnp.float32),
    scratch_types=[
        pltpu.VMEM((BUCKET, 32), jnp.float32),
        pltpu.VMEM((1, NB * 16), jnp.int32),
        pltpu.VMEM((MBLK // 128, 128), jnp.int32),
        pltpu.VMEM((MBLK + 16,), jnp.int32),
        pltpu.VMEM((MBLK,), jnp.int32),
        pltpu.VMEM((MBLK, 32), jnp.float32),
        pltpu.SemaphoreType.DMA,
    ],
)(_max_body)


_RB = 5000
_GRID = N // _RB


def _dense0_body(x_ref, p0_ref, p1_ref, wr_ref, b_ref, wo_ref,
                 ha_ref, hb_ref, rec_ref):
    s = p0_ref[:, 0:1] + p1_ref[:, 0:1]
    deg = p0_ref[:, 1:2] + p1_ref[:, 1:2]
    h = (s * wr_ref[0:1, :] + b_ref[...][None, :]
         + x_ref[...] * wo_ref[0:1, :])
    h = jnp.maximum(h, 0.0)
    ha_ref[...] = h[:, :16]
    hb_ref[...] = h[:, 16:]
    rec_ref[...] = 1.0 / jnp.maximum(deg, 1.0)


def _dense0(x1, p0, p1, wr, b, wo):
    return pl.pallas_call(
        _dense0_body,
        grid=(_GRID,),
        in_specs=[
            pl.BlockSpec((_RB, 1), lambda i: (i, 0)),
            pl.BlockSpec((_RB, 8), lambda i: (i, 0)),
            pl.BlockSpec((_RB, 8), lambda i: (i, 0)),
            pl.BlockSpec((1, 32), lambda i: (0, 0)),
            pl.BlockSpec((32,), lambda i: (0,)),
            pl.BlockSpec((1, 32), lambda i: (0, 0)),
        ],
        out_specs=[
            pl.BlockSpec((_RB, 16), lambda i: (i, 0)),
            pl.BlockSpec((_RB, 16), lambda i: (i, 0)),
            pl.BlockSpec((_RB, 1), lambda i: (i, 0)),
        ],
        out_shape=[
            jax.ShapeDtypeStruct((N, 16), jnp.float32),
            jax.ShapeDtypeStruct((N, 16), jnp.float32),
            jax.ShapeDtypeStruct((N, 1), jnp.float32),
        ],
    )(x1, p0, p1, wr, b, wo)


def _dense_mid_body(use_mean, split_s, outs, sa_ref, sb_ref, ha_ref, hb_ref,
                    wr_ref, b_ref, wo_ref, rec_ref, *out_refs):
    if split_s:
        s = jnp.concatenate([sa_ref[...], sb_ref[...]], axis=1)
    else:
        s = sa_ref[...]
    if use_mean:
        s = s * rec_ref[...]
    h = jnp.concatenate([ha_ref[...], hb_ref[...]], axis=1)
    z = (jnp.dot(s, wr_ref[...], preferred_element_type=jnp.float32)
         + b_ref[...][None, :]
         + jnp.dot(h, wo_ref[...], preferred_element_type=jnp.float32))
    z = jnp.maximum(z, 0.0)
    i = 0
    if "halves" in outs:
        out_refs[i][...] = z[:, :16]
        out_refs[i + 1][...] = z[:, 16:]
        i += 2
    if "full" in outs:
        out_refs[i][...] = z


def _dense_mid(s_parts, ha, hb, wr, b, wo, rec, use_mean, outs=("halves",)):
    split_s = len(s_parts) == 2
    if split_s:
        s_specs = [pl.BlockSpec((_RB, 16), lambda i: (i, 0)),
                   pl.BlockSpec((_RB, 16), lambda i: (i, 0))]
        sa, sb = s_parts
    else:
        s_specs = [pl.BlockSpec((_RB, 32), lambda i: (i, 0)),
                   pl.BlockSpec((_RB, 32), lambda i: (i, 0))]
        sa = sb = s_parts[0]
    out_specs, out_shape = [], []
    if "halves" in outs:
        out_specs += [pl.BlockSpec((_RB, 16), lambda i: (i, 0)),
                      pl.BlockSpec((_RB, 16), lambda i: (i, 0))]
        out_shape += [jax.ShapeDtypeStruct((N, 16), jnp.float32),
                      jax.ShapeDtypeStruct((N, 16), jnp.float32)]
    if "full" in outs:
        out_specs += [pl.BlockSpec((_RB, 32), lambda i: (i, 0))]
        out_shape += [jax.ShapeDtypeStruct((N, 32), jnp.float32)]
    return pl.pallas_call(
        functools.partial(_dense_mid_body, use_mean, split_s, outs),
        grid=(_GRID,),
        in_specs=s_specs + [
            pl.BlockSpec((_RB, 16), lambda i: (i, 0)),
            pl.BlockSpec((_RB, 16), lambda i: (i, 0)),
            pl.BlockSpec((32, 32), lambda i: (0, 0)),
            pl.BlockSpec((32,), lambda i: (0,)),
            pl.BlockSpec((32, 32), lambda i: (0, 0)),
            pl.BlockSpec((_RB, 1), lambda i: (i, 0)),
        ],
        out_specs=out_specs,
        out_shape=out_shape,
    )(sa, sb, ha, hb, wr, b, wo, rec)


def _dense_final_body(s_ref, ha_ref, hb_ref, wr_ref, b_ref, wo_ref, o_ref):
    h = jnp.concatenate([ha_ref[...], hb_ref[...]], axis=1)
    z = (jnp.dot(s_ref[...], wr_ref[...], preferred_element_type=jnp.float32)
         + b_ref[...][None, :]
         + jnp.dot(h, wo_ref[...], preferred_element_type=jnp.float32))
    o_ref[...] = jax.nn.sigmoid(z)


def _dense_final(s, ha, hb, wr, b, wo):
    return pl.pallas_call(
        _dense_final_body,
        grid=(_GRID,),
        in_specs=[
            pl.BlockSpec((_RB, 32), lambda i: (i, 0)),
            pl.BlockSpec((_RB, 16), lambda i: (i, 0)),
            pl.BlockSpec((_RB, 16), lambda i: (i, 0)),
            pl.BlockSpec((32, 1), lambda i: (0, 0)),
            pl.BlockSpec((1,), lambda i: (0,)),
            pl.BlockSpec((32, 1), lambda i: (0, 0)),
        ],
        out_specs=pl.BlockSpec((_RB, 1), lambda i: (i, 0)),
        out_shape=jax.ShapeDtypeStruct((N, 1), jnp.float32),
    )(s, ha, hb, wr, b, wo)


def kernel(x, edge_index, W_rel0, b_rel0, W_root0, W_rel1, b_rel1, W_root1,
           W_rel2, b_rel2, W_root2, W_rel3, b_rel3, W_root3, W_rel4, b_rel4,
           W_root4, W_rel5, b_rel5, W_root5):
    src2d = edge_index[0].reshape(ROWS, 128)
    dst2d = edge_index[1].reshape(ROWS, 128)
    tab0 = jnp.concatenate(
        [x[:, None], jnp.ones((N, 1), jnp.float32),
         jnp.zeros((N, 6), jnp.float32)], axis=1)
    zeros16 = jnp.zeros((N, 16), jnp.float32)
    zeros8 = jnp.zeros((N, 8), jnp.float32)

    part, hist = _l0_kernel(tab0, src2d, dst2d, zeros8)
    bpack, btot = _b2_kernel(src2d, dst2d, hist)

    h1a, h1b, rec = _dense0(x.reshape(N, 1), part[0], part[1],
                            W_rel0, b_rel0, W_root0)

    s1a, s1b = _sum_kernel(h1a, h1b, src2d, dst2d, zeros16)
    h2a, h2b = _dense_mid((s1a, s1b), h1a, h1b, W_rel1, b_rel1, W_root1, rec,
                          use_mean=True)

    s2a, s2b = _sum_kernel(h2a, h2b, src2d, dst2d, zeros16)
    h3a, h3b = _dense_mid((s2a, s2b), h2a, h2b, W_rel2, b_rel2, W_root2, rec,
                          use_mean=False)

    s3a, s3b = _sum_kernel(h3a, h3b, src2d, dst2d, zeros16)
    h4a, h4b, h4 = _dense_mid((s3a, s3b), h3a, h3b, W_rel3, b_rel3, W_root3,
                              rec, use_mean=True, outs=("halves", "full"))

    m4 = _max_kernel(h4, bpack, btot)
    h5a, h5b, h5 = _dense_mid((m4,), h4a, h4b, W_rel4, b_rel4, W_root4, rec,
                              use_mean=False, outs=("halves", "full"))

    m5 = _max_kernel(h5, bpack, btot)
    out = _dense_final(m5, h5a, h5b, W_rel5, b_rel5, W_root5)
    return out.reshape(N)


# speedup vs baseline: 8.8876x; 1.2074x over previous
import functools

import jax
import jax.numpy as jnp
from jax import lax
from jax.experimental import pallas as pl
from jax.experimental.pallas import tpu as pltpu
from jax.experimental.pallas import tpu_sc as plsc

N = 100000
E = 1600000
ROWS = E // 128
UNITS = ROWS // 16
REM_ROW = UNITS * 16
NT = 16
NC = 2
NW = NT * NC
NB = 32
BUCKET = 3128
LASTB = N - 31 * BUCKET
SLICE = 2 * BUCKET
LAST = N - 15 * SLICE
EP = E + 512
PBITS = 12

_mesh = plsc.VectorSubcoreMesh(core_axis_name="c", subcore_axis_name="s")
_SC_PARAMS = pltpu.CompilerParams(needs_layout_passes=False,
                                  use_tc_tiling_on_sc=False)


def _bucket_of(d):
    return (d.astype(jnp.float32) / float(BUCKET)).astype(jnp.int32)


def _tile_rows(t):
    return t * SLICE, (LAST if t == NT - 1 else SLICE)


def _copy_tile_slice(t, src, dst):
    for ts in range(NT):
        start, size = _tile_rows(ts)

        @pl.when(t == ts)
        def _(start=start, size=size):
            pltpu.sync_copy(src.at[pl.ds(start, size)], dst.at[pl.ds(start, size)])


def _unit_range(i, n):
    return (i * UNITS) // n, ((i + 1) * UNITS) // n


def _l0_body(tab0, src2d, dst2d, zeros8, part, hist, acc, sbufs, dbufs, msgsA,
             msgsB, histv, gsem, ssem):
    c = lax.axis_index("c")
    t = lax.axis_index("s")
    w = t * NC + c
    iota = lax.iota(jnp.int32, 16)
    ones = jnp.ones((16,), jnp.int32)

    _copy_tile_slice(t, zeros8, acc)

    def zh(i, _):
        histv[0, pl.ds(i * 16, 16)] = jnp.zeros((16,), jnp.int32)
        return _

    lax.fori_loop(0, NB, zh, 0)
    plsc.subcore_barrier()

    def hist_rows(nrows):
        def hrow(j, _):
            def hvec(v, _):
                d = dbufs[j, pl.ds(v * 16, 16)]
                b = _bucket_of(d)
                plsc.addupdate_scatter(histv.at[0], [b * 16 + iota], ones)
                return _

            return lax.fori_loop(0, 8, hvec, _)

        lax.fori_loop(0, nrows, hrow, 0)

    def do_batch(r, nrows):
        pltpu.sync_copy(src2d.at[pl.ds(r, nrows)], sbufs.at[pl.ds(0, nrows)])
        pltpu.sync_copy(dst2d.at[pl.ds(r, nrows)], dbufs.at[pl.ds(0, nrows)])
        na = min(8, nrows)
        ga = [pltpu.async_copy(tab0.at[sbufs.at[j]], msgsA.at[j], gsem)
              for j in range(na)]
        for cp in ga:
            cp.wait()
        sa = [pltpu.async_copy(msgsA.at[j], acc.at[dbufs.at[j]], ssem,
                               add=True)
              for j in range(na)]
        sb = []
        if nrows > 8:
            gb = [pltpu.async_copy(tab0.at[sbufs.at[8 + j]], msgsB.at[j], gsem)
                  for j in range(nrows - 8)]
            for cp in gb:
                cp.wait()
            sb = [pltpu.async_copy(msgsB.at[j], acc.at[dbufs.at[8 + j]], ssem,
                                   add=True)
                  for j in range(nrows - 8)]
        hist_rows(nrows)
        for cp in sa + sb:
            cp.wait()

    lo_u, hi_u = _unit_range(w, NW)

    def batch(k, _):
        do_batch(k * 16, 16)
        return _

    lax.fori_loop(lo_u, hi_u, batch, 0)

    @pl.when(w == NW - 1)
    def _rem():
        do_batch(REM_ROW, 4)

    plsc.subcore_barrier()
    for ts in range(NT):
        start, size = _tile_rows(ts)

        @pl.when(t == ts)
        def _(start=start, size=size):
            pltpu.sync_copy(acc.at[pl.ds(start, size)],
                            part.at[c, pl.ds(start, size)])
    pltpu.sync_copy(histv, hist.at[w])


_l0_kernel = functools.partial(
    pl.kernel,
    mesh=_mesh,
    compiler_params=_SC_PARAMS,
    out_type=[
        jax.ShapeDtypeStruct((NC, N, 8), jnp.float32),
        jax.ShapeDtypeStruct((NW, 1, NB * 16), jnp.int32),
    ],
    scratch_types=[
        pltpu.VMEM_SHARED((N, 8), jnp.float32),
        pltpu.VMEM((16, 128), jnp.int32),
        pltpu.VMEM((16, 128), jnp.int32),
        pltpu.VMEM((8, 128, 8), jnp.float32),
        pltpu.VMEM((8, 128, 8), jnp.float32),
        pltpu.VMEM((1, NB * 16), jnp.int32),
        pltpu.SemaphoreType.DMA,
        pltpu.SemaphoreType.DMA,
    ],
)(_l0_body)


SCAP = 1600640
STILE = SCAP // NT


def _b2_body(src2d, dst2d, hist, bp0, bp1, btot, stage, histv, curs, sbufs,
             dbufs, packv, posv, totv, s1):
    c = lax.axis_index("c")
    t = lax.axis_index("s")
    iota = lax.iota(jnp.int32, 16)
    zeros16i = jnp.zeros((16,), jnp.int32)

    for w2 in range(NW):
        pltpu.sync_copy(hist.at[w2, 0, pl.ds(c * 256, 256)], histv.at[w2])

    base_acc = jnp.int32(0)
    for b in range(16):
        def inner(t2, carry, b=b):
            pre, tot = carry
            row = (histv[2 * t2, pl.ds(b * 16, 16)]
                   + histv[2 * t2 + 1, pl.ds(b * 16, 16)])
            pre = pre + jnp.where(t2 < t, row, zeros16i)
            tot = tot + row
            return (pre, tot)

        pre, tot = lax.fori_loop(0, NT, inner, (zeros16i, zeros16i))
        own = (histv[2 * t, pl.ds(b * 16, 16)]
               + histv[2 * t + 1, pl.ds(b * 16, 16)])
        ownprefix = plsc.cumsum(own) - own
        curs[pl.ds(b * 16, 16)] = base_acc + jnp.sum(pre) + ownprefix
        base_acc = base_acc + jnp.sum(tot)

        @pl.when(t == 0)
        def _(b=b, tot=tot):
            totv[0, pl.ds(b * 16, 16)] = tot

    @pl.when(t == 0)
    def _tot():
        pltpu.sync_copy(totv, btot.at[:, pl.ds(c * 256, 256)])

    def do_batch(r, nrows):
        pltpu.sync_copy(src2d.at[pl.ds(r, nrows)], sbufs.at[pl.ds(0, nrows)])
        pltpu.sync_copy(dst2d.at[pl.ds(r, nrows)], dbufs.at[pl.ds(0, nrows)])
        cps = []
        for j in range(nrows):
            for v in range(8):
                d = dbufs[j, pl.ds(v * 16, 16)]
                sv = sbufs[j, pl.ds(v * 16, 16)]
                b = _bucket_of(d)
                bl = b - c * 16
                own = (bl >= 0) & (bl < 16)
                idx = jnp.where(own, bl * 16 + iota, 0)
                pos = plsc.load_gather(curs, [idx])
                plsc.addupdate_scatter(curs, [idx],
                                       jnp.where(own, 1, 0).astype(jnp.int32))
                posv[j, pl.ds(v * 16, 16)] = jnp.where(own, pos, SCAP - 8)
                packv[j, pl.ds(v * 16, 16)] = ((sv << PBITS)
                                               | (d - b * BUCKET))
            cps.append(pltpu.async_copy(packv.at[j], stage.at[posv.at[j]], s1))
        for cp in cps:
            cp.wait()

    lo_u, hi_u = _unit_range(t, NT)

    def batch(k, _):
        do_batch(k * 16, 16)
        return _

    lax.fori_loop(lo_u, hi_u, batch, 0)

    @pl.when(t == NT - 1)
    def _rem():
        do_batch(REM_ROW, 4)

    plsc.subcore_barrier()

    @pl.when(c == 0)
    def _w0():
        pltpu.sync_copy(stage.at[pl.ds(t * STILE, STILE)],
                        bp0.at[pl.ds(t * STILE, STILE)])

    @pl.when(c == 1)
    def _w1():
        pltpu.sync_copy(stage.at[pl.ds(t * STILE, STILE)],
                        bp1.at[pl.ds(t * STILE, STILE)])


_b2_kernel = functools.partial(
    pl.kernel,
    mesh=_mesh,
    compiler_params=_SC_PARAMS,
    out_type=[
        jax.ShapeDtypeStruct((SCAP,), jnp.int32),
        jax.ShapeDtypeStruct((SCAP,), jnp.int32),
        jax.ShapeDtypeStruct((1, NB * 16), jnp.int32),
    ],
    scratch_types=[
        pltpu.VMEM_SHARED((SCAP,), jnp.int32),
        pltpu.VMEM((NW, 256), jnp.int32),
        pltpu.VMEM((256,), jnp.int32),
        pltpu.VMEM((16, 128), jnp.int32),
        pltpu.VMEM((16, 128), jnp.int32),
        pltpu.VMEM((16, 128), jnp.int32),
        pltpu.VMEM((16, 128), jnp.int32),
        pltpu.VMEM((1, 256), jnp.int32),
        pltpu.SemaphoreType.DMA,
    ],
)(_b2_body)


def _sum_body(hA, hB, src2d, dst2d, zeros16, sA, sB, acc, sbufs, dbufs, msgsA,
              msgsB, gsem, ssem):
    c = lax.axis_index("c")
    t = lax.axis_index("s")

    _copy_tile_slice(t, zeros16, acc)
    plsc.subcore_barrier()

    def run(tab):
        def do_batch(r, nrows):
            pltpu.sync_copy(src2d.at[pl.ds(r, nrows)], sbufs.at[pl.ds(0, nrows)])
            pltpu.sync_copy(dst2d.at[pl.ds(r, nrows)], dbufs.at[pl.ds(0, nrows)])
            bufs = [msgsA, msgsB]
            pend = [[], []]
            for wv in range(nrows // 4):
                bi = wv % 2
                for cp in pend[bi]:
                    cp.wait()
                g = [pltpu.async_copy(tab.at[sbufs.at[wv * 4 + j]],
                                      bufs[bi].at[j], gsem)
                     for j in range(4)]
                for cp in g:
                    cp.wait()
                pend[bi] = [pltpu.async_copy(bufs[bi].at[j],
                                             acc.at[dbufs.at[wv * 4 + j]],
                                             ssem, add=True)
                            for j in range(4)]
            for cp in pend[0] + pend[1]:
                cp.wait()

        lo_u, hi_u = _unit_range(t, NT)

        def batch(k, _):
            do_batch(k * 16, 16)
            return _

        lax.fori_loop(lo_u, hi_u, batch, 0)

        @pl.when(t == NT - 1)
        def _rem():
            do_batch(REM_ROW, 4)

    @pl.when(c == 0)
    def _a():
        run(hA)

    @pl.when(c == 1)
    def _b():
        run(hB)

    plsc.subcore_barrier()
    for ts in range(NT):
        start, size = _tile_rows(ts)

        @pl.when(t == ts)
        def _(start=start, size=size):
            @pl.when(c == 0)
            def _wa():
                pltpu.sync_copy(acc.at[pl.ds(start, size)],
                                sA.at[pl.ds(start, size)])

            @pl.when(c == 1)
            def _wb():
                pltpu.sync_copy(acc.at[pl.ds(start, size)],
                                sB.at[pl.ds(start, size)])


_sum_kernel = functools.partial(
    pl.kernel,
    mesh=_mesh,
    compiler_params=_SC_PARAMS,
    out_type=[
        jax.ShapeDtypeStruct((N, 16), jnp.float32),
        jax.ShapeDtypeStruct((N, 16), jnp.float32),
    ],
    scratch_types=[
        pltpu.VMEM_SHARED((N, 16), jnp.float32),
        pltpu.VMEM((16, 128), jnp.int32),
        pltpu.VMEM((16, 128), jnp.int32),
        pltpu.VMEM((4, 128, 16), jnp.float32),
        pltpu.VMEM((4, 128, 16), jnp.float32),
        pltpu.SemaphoreType.DMA,
        pltpu.SemaphoreType.DMA,
    ],
)(_sum_body)


MBLK = 512


def _max_body(h32, bp0, bp1, btot, m32, acc, totv, sbufs, dlv, pbuf, msgs,
              gsem):
    c = lax.axis_index("c")
    t = lax.axis_index("s")
    w = t * NC + c
    mask = jnp.int32((1 << PBITS) - 1)
    iota = lax.iota(jnp.int32, 16)

    pltpu.sync_copy(btot, totv)

    def fold(q, carry):
        base, cnt, half0 = carry
        v = jnp.sum(totv[0, pl.ds(q * 16, 16)])
        base = base + jnp.where(q < w, v, 0)
        cnt = cnt + jnp.where(q == w, v, 0)
        half0 = half0 + jnp.where(q < 16, v, 0)
        return (base, cnt, half0)

    base, cnt, half0 = lax.fori_loop(
        0, NB, fold, (jnp.int32(0), jnp.int32(0), jnp.int32(0)))
    in0 = w < 16
    lbase = base - jnp.where(in0, 0, half0)

    def foldv(q, acc2):
        v = jnp.sum(totv[0, pl.ds(q * 16, 16)])
        return acc2 + jnp.where(q >= 16, v, 0)

    vend = jnp.where(in0, half0, lax.fori_loop(0, NB, foldv, jnp.int32(0)))

    def za(i, _):
        acc[i, pl.ds(0, 16)] = jnp.zeros((16,), jnp.float32)
        acc[i, pl.ds(16, 16)] = jnp.zeros((16,), jnp.float32)
        return _

    lax.fori_loop(0, BUCKET, za, 0)

    blk0 = lbase // MBLK
    off0 = lbase - blk0 * MBLK
    nblk = (off0 + cnt + MBLK - 1) // MBLK

    def run(bp):
        def blk(k, _):
            g = (blk0 + k) * MBLK
            pltpu.sync_copy(bp.at[pl.ds(g, MBLK)], pbuf)

            def unpack(j2, _):
                def up(v, _):
                    p = pbuf[pl.ds(j2 * 128 + v * 16, 16)]
                    gpos = g + j2 * 128 + v * 16 + iota
                    src = lax.shift_right_logical(p, PBITS)
                    sbufs[j2, pl.ds(v * 16, 16)] = jnp.where(gpos < vend,
                                                             src, 0)
                    dlv[pl.ds(j2 * 128 + v * 16, 16)] = p & mask
                    return _

                return lax.fori_loop(0, 8, up, _)

            lax.fori_loop(0, MBLK // 128, unpack, 0)
            cps = [pltpu.async_copy(h32.at[sbufs.at[j]],
                                    msgs.at[pl.ds(j * 128, 128)], gsem)
                   for j in range(MBLK // 128)]
            for cp in cps:
                cp.wait()
            e_lo = jnp.maximum(lbase - g, 0)
            e_hi = jnp.minimum(lbase + cnt - g, MBLK)

            def one_edge(e):
                dl = dlv[pl.ds(e, 16)][0]
                acc[dl, pl.ds(0, 16)] = jnp.maximum(acc[dl, pl.ds(0, 16)],
                                                    msgs[e, pl.ds(0, 16)])
                acc[dl, pl.ds(16, 16)] = jnp.maximum(acc[dl, pl.ds(16, 16)],
                                                     msgs[e, pl.ds(16, 16)])

            def body8(e):
                for u in range(8):
                    one_edge(e + u)
                return e + 8

            e_mid = lax.while_loop(lambda e: e + 8 <= e_hi, body8, e_lo)

            def edge(e, _):
                one_edge(e)
                return _

            lax.fori_loop(e_mid, e_hi, edge, 0)
            return _

        lax.fori_loop(0, nblk, blk, 0)

    @pl.when(in0)
    def _r0():
        run(bp0)

    @pl.when(jnp.logical_not(in0))
    def _r1():
        run(bp1)

    @pl.when(w < NW - 1)
    def _wr():
        pltpu.sync_copy(acc.at[pl.ds(0, BUCKET)],
                        m32.at[pl.ds(w * BUCKET, BUCKET)])

    @pl.when(w == NW - 1)
    def _wl():
        pltpu.sync_copy(acc.at[pl.ds(0, LASTB)],
                        m32.at[pl.ds((NW - 1) * BUCKET, LASTB)])


_max_kernel = functools.partial(
    pl.kernel,
    mesh=_mesh,
    compiler_params=_SC_PARAMS,
    out_type=jax.ShapeDtypeStruct((N, 32), jnp.float32),
    scratch_types=[
        pltpu.VMEM((BUCKET, 32), jnp.float32),
        pltpu.VMEM((1, NB * 16), jnp.int32),
        pltpu.VMEM((MBLK // 128, 128), jnp.int32),
        pltpu.VMEM((MBLK + 16,), jnp.int32),
        pltpu.VMEM((MBLK,), jnp.int32),
        pltpu.VMEM((MBLK, 32), jnp.float32),
        pltpu.SemaphoreType.DMA,
    ],
)(_max_body)


_RB = 5000
_GRID = N // _RB


def _dense0_body(x_ref, p0_ref, p1_ref, wr_ref, b_ref, wo_ref,
                 ha_ref, hb_ref, rec_ref):
    s = p0_ref[:, 0:1] + p1_ref[:, 0:1]
    deg = p0_ref[:, 1:2] + p1_ref[:, 1:2]
    h = (s * wr_ref[0:1, :] + b_ref[...][None, :]
         + x_ref[...] * wo_ref[0:1, :])
    h = jnp.maximum(h, 0.0)
    ha_ref[...] = h[:, :16]
    hb_ref[...] = h[:, 16:]
    rec_ref[...] = 1.0 / jnp.maximum(deg, 1.0)


def _dense0(x1, p0, p1, wr, b, wo):
    return pl.pallas_call(
        _dense0_body,
        grid=(_GRID,),
        in_specs=[
            pl.BlockSpec((_RB, 1), lambda i: (i, 0)),
            pl.BlockSpec((_RB, 8), lambda i: (i, 0)),
            pl.BlockSpec((_RB, 8), lambda i: (i, 0)),
            pl.BlockSpec((1, 32), lambda i: (0, 0)),
            pl.BlockSpec((32,), lambda i: (0,)),
            pl.BlockSpec((1, 32), lambda i: (0, 0)),
        ],
        out_specs=[
            pl.BlockSpec((_RB, 16), lambda i: (i, 0)),
            pl.BlockSpec((_RB, 16), lambda i: (i, 0)),
            pl.BlockSpec((_RB, 1), lambda i: (i, 0)),
        ],
        out_shape=[
            jax.ShapeDtypeStruct((N, 16), jnp.float32),
            jax.ShapeDtypeStruct((N, 16), jnp.float32),
            jax.ShapeDtypeStruct((N, 1), jnp.float32),
        ],
    )(x1, p0, p1, wr, b, wo)


def _dense_mid_body(use_mean, split_s, outs, sa_ref, sb_ref, ha_ref, hb_ref,
                    wr_ref, b_ref, wo_ref, rec_ref, *out_refs):
    if split_s:
        s = jnp.concatenate([sa_ref[...], sb_ref[...]], axis=1)
    else:
        s = sa_ref[...]
    if use_mean:
        s = s * rec_ref[...]
    h = jnp.concatenate([ha_ref[...], hb_ref[...]], axis=1)
    z = (jnp.dot(s, wr_ref[...], preferred_element_type=jnp.float32)
         + b_ref[...][None, :]
         + jnp.dot(h, wo_ref[...], preferred_element_type=jnp.float32))
    z = jnp.maximum(z, 0.0)
    i = 0
    if "halves" in outs:
        out_refs[i][...] = z[:, :16]
        out_refs[i + 1][...] = z[:, 16:]
        i += 2
    if "full" in outs:
        out_refs[i][...] = z


def _dense_mid(s_parts, ha, hb, wr, b, wo, rec, use_mean, outs=("halves",)):
    split_s = len(s_parts) == 2
    if split_s:
        s_specs = [pl.BlockSpec((_RB, 16), lambda i: (i, 0)),
                   pl.BlockSpec((_RB, 16), lambda i: (i, 0))]
        sa, sb = s_parts
    else:
        s_specs = [pl.BlockSpec((_RB, 32), lambda i: (i, 0)),
                   pl.BlockSpec((_RB, 32), lambda i: (i, 0))]
        sa = sb = s_parts[0]
    out_specs, out_shape = [], []
    if "halves" in outs:
        out_specs += [pl.BlockSpec((_RB, 16), lambda i: (i, 0)),
                      pl.BlockSpec((_RB, 16), lambda i: (i, 0))]
        out_shape += [jax.ShapeDtypeStruct((N, 16), jnp.float32),
                      jax.ShapeDtypeStruct((N, 16), jnp.float32)]
    if "full" in outs:
        out_specs += [pl.BlockSpec((_RB, 32), lambda i: (i, 0))]
        out_shape += [jax.ShapeDtypeStruct((N, 32), jnp.float32)]
    return pl.pallas_call(
        functools.partial(_dense_mid_body, use_mean, split_s, outs),
        grid=(_GRID,),
        in_specs=s_specs + [
            pl.BlockSpec((_RB, 16), lambda i: (i, 0)),
            pl.BlockSpec((_RB, 16), lambda i: (i, 0)),
            pl.BlockSpec((32, 32), lambda i: (0, 0)),
            pl.BlockSpec((32,), lambda i: (0,)),
            pl.BlockSpec((32, 32), lambda i: (0, 0)),
            pl.BlockSpec((_RB, 1), lambda i: (i, 0)),
        ],
        out_specs=out_specs,
        out_shape=out_shape,
    )(sa, sb, ha, hb, wr, b, wo, rec)


def _dense_final_body(s_ref, ha_ref, hb_ref, wr_ref, b_ref, wo_ref, o_ref):
    h = jnp.concatenate([ha_ref[...], hb_ref[...]], axis=1)
    z = (jnp.dot(s_ref[...], wr_ref[...], preferred_element_type=jnp.float32)
         + b_ref[...][None, :]
         + jnp.dot(h, wo_ref[...], preferred_element_type=jnp.float32))
    o_ref[...] = jax.nn.sigmoid(z)


def _dense_final(s, ha, hb, wr, b, wo):
    return pl.pallas_call(
        _dense_final_body,
        grid=(_GRID,),
        in_specs=[
            pl.BlockSpec((_RB, 32), lambda i: (i, 0)),
            pl.BlockSpec((_RB, 16), lambda i: (i, 0)),
            pl.BlockSpec((_RB, 16), lambda i: (i, 0)),
            pl.BlockSpec((32, 1), lambda i: (0, 0)),
            pl.BlockSpec((1,), lambda i: (0,)),
            pl.BlockSpec((32, 1), lambda i: (0, 0)),
        ],
        out_specs=pl.BlockSpec((_RB, 1), lambda i: (i, 0)),
        out_shape=jax.ShapeDtypeStruct((N, 1), jnp.float32),
    )(s, ha, hb, wr, b, wo)


def kernel(x, edge_index, W_rel0, b_rel0, W_root0, W_rel1, b_rel1, W_root1,
           W_rel2, b_rel2, W_root2, W_rel3, b_rel3, W_root3, W_rel4, b_rel4,
           W_root4, W_rel5, b_rel5, W_root5):
    src2d = edge_index[0].reshape(ROWS, 128)
    dst2d = edge_index[1].reshape(ROWS, 128)
    tab0 = jnp.concatenate(
        [x[:, None], jnp.ones((N, 1), jnp.float32),
         jnp.zeros((N, 6), jnp.float32)], axis=1)
    zeros16 = jnp.zeros((N, 16), jnp.float32)
    zeros8 = jnp.zeros((N, 8), jnp.float32)

    part, hist = _l0_kernel(tab0, src2d, dst2d, zeros8)
    bp0, bp1, btot = _b2_kernel(src2d, dst2d, hist)

    h1a, h1b, rec = _dense0(x.reshape(N, 1), part[0], part[1],
                            W_rel0, b_rel0, W_root0)

    s1a, s1b = _sum_kernel(h1a, h1b, src2d, dst2d, zeros16)
    h2a, h2b = _dense_mid((s1a, s1b), h1a, h1b, W_rel1, b_rel1, W_root1, rec,
                          use_mean=True)

    s2a, s2b = _sum_kernel(h2a, h2b, src2d, dst2d, zeros16)
    h3a, h3b = _dense_mid((s2a, s2b), h2a, h2b, W_rel2, b_rel2, W_root2, rec,
                          use_mean=False)

    s3a, s3b = _sum_kernel(h3a, h3b, src2d, dst2d, zeros16)
    h4a, h4b, h4 = _dense_mid((s3a, s3b), h3a, h3b, W_rel3, b_rel3, W_root3,
                              rec, use_mean=True, outs=("halves", "full"))

    m4 = _max_kernel(h4, bp0, bp1, btot)
    h5a, h5b, h5 = _dense_mid((m4,), h4a, h4b, W_rel4, b_rel4, W_root4, rec,
                              use_mean=False, outs=("halves", "full"))

    m5 = _max_kernel(h5, bp0, bp1, btot)
    out = _dense_final(m5, h5a, h5b, W_rel5, b_rel5, W_root5)
    return out.reshape(N)
